# speedup vs baseline: 11.8229x; 11.8229x over previous
import functools

import jax
import jax.numpy as jnp
from jax import lax
from jax.experimental import pallas as pl
from jax.experimental.pallas import tpu as pltpu

_B = 4
_N = 20000
_ROWS = 160
_LANES = 128
_PADN = _ROWS * _LANES
_MAX_DET = 100
_IOU_T = 0.2
_SCORE_T = 0.2
_NEG = float("-inf")


def _nms_body(a0, a1, a2, a3, dy, dx, dh, dw, cl,
              ox1, oy1, ox2, oy2, osc,
              x1r, y1r, x2r, y2r, arr, ssr, sor,
              *, wclip, hclip):
    a0v = a0[...]
    a1v = a1[...]
    a2v = a2[...]
    a3v = a3[...]
    ya = ((a0v + a2v) / 2.0)[None]
    xa = ((a1v + a3v) / 2.0)[None]
    ha = (a2v - a0v)[None]
    wa = (a3v - a1v)[None]
    h = jnp.exp(dh[...]) * ha
    w = jnp.exp(dw[...]) * wa
    yc = dy[...] * ha + ya
    xc = dx[...] * wa + xa
    xmin = jnp.clip(xc - w / 2.0, 0.0, wclip)
    ymin = jnp.clip(yc - h / 2.0, 0.0, hclip)
    xmax = jnp.clip(xc + w / 2.0, 0.0, wclip)
    ymax = jnp.clip(yc + h / 2.0, 0.0, hclip)
    s_orig = jax.nn.sigmoid(cl[...])
    s0 = jnp.where(s_orig > _SCORE_T, s_orig, _NEG)
    x1r[...] = xmin
    y1r[...] = ymin
    x2r[...] = xmax
    y2r[...] = ymax
    arr[...] = (xmax - xmin) * (ymax - ymin)
    ssr[...] = s0
    sor[...] = s_orig

    ii = (lax.broadcasted_iota(jnp.int32, (_ROWS, _LANES), 0) * _LANES
          + lax.broadcasted_iota(jnp.int32, (_ROWS, _LANES), 1))
    lane = lax.broadcasted_iota(jnp.int32, (1, _LANES), 1)
    big = jnp.int32(2**30)

    for b in range(_B):
        x1 = x1r[b]
        y1 = y1r[b]
        x2 = x2r[b]
        y2 = y2r[b]
        ar = arr[b]
        so = sor[b]

        def body(i, _, b=b, x1=x1, y1=y1, x2=x2, y2=y2, ar=ar, so=so):
            s = ssr[b]
            m = jnp.max(s)
            idx = jnp.min(jnp.where(s == m, ii, big))
            selm = ii == idx
            bx1 = jnp.max(jnp.where(selm, x1, _NEG))
            by1 = jnp.max(jnp.where(selm, y1, _NEG))
            bx2 = jnp.max(jnp.where(selm, x2, _NEG))
            by2 = jnp.max(jnp.where(selm, y2, _NEG))
            bar = jnp.max(jnp.where(selm, ar, _NEG))
            bsc = jnp.max(jnp.where(selm, so, _NEG))
            xx1 = jnp.maximum(bx1, x1)
            yy1 = jnp.maximum(by1, y1)
            xx2 = jnp.minimum(bx2, x2)
            yy2 = jnp.minimum(by2, y2)
            inter = jnp.maximum(xx2 - xx1, 0.0) * jnp.maximum(yy2 - yy1, 0.0)
            union = ar + bar - inter
            iou = inter / jnp.maximum(union, 1e-8)
            ssr[b] = jnp.where(iou > _IOU_T, _NEG, s)
            valid = bsc > _SCORE_T
            hit = lane == i
            ox1[b] = jnp.where(hit, jnp.where(valid, bx1, 0.0), ox1[b])
            oy1[b] = jnp.where(hit, jnp.where(valid, by1, 0.0), oy1[b])
            ox2[b] = jnp.where(hit, jnp.where(valid, bx2, 0.0), ox2[b])
            oy2[b] = jnp.where(hit, jnp.where(valid, by2, 0.0), oy2[b])
            osc[b] = jnp.where(hit, jnp.where(valid, bsc, 0.0), osc[b])
            return 0

        lax.fori_loop(0, _MAX_DET, body, 0)


@jax.jit
def kernel(imgs, anchors, regression, classification):
    hc = float(imgs.shape[2] - 1)
    wc = float(imgs.shape[3] - 1)
    pad = _PADN - _N
    anc = jnp.pad(anchors, ((0, pad), (0, 0)))
    reg = jnp.pad(regression, ((0, 0), (0, pad), (0, 0)))
    cls = jnp.pad(classification[..., 0], ((0, 0), (0, pad)),
                  constant_values=-1e9)
    a0, a1, a2, a3 = [anc[:, i].reshape(_ROWS, _LANES) for i in range(4)]
    dy, dx, dh, dw = [reg[..., i].reshape(_B, _ROWS, _LANES) for i in range(4)]
    cl = cls.reshape(_B, _ROWS, _LANES)

    outs = pl.pallas_call(
        functools.partial(_nms_body, wclip=wc, hclip=hc),
        out_shape=[jax.ShapeDtypeStruct((_B, 1, _LANES), jnp.float32)] * 5,
        scratch_shapes=[pltpu.VMEM((_B, _ROWS, _LANES), jnp.float32)] * 7,
    )(a0, a1, a2, a3, dy, dx, dh, dw, cl)
    ox1, oy1, ox2, oy2, osc = outs
    out = jnp.stack([ox1, oy1, ox2, oy2, osc], axis=-1)
    return out[:, 0, :_MAX_DET, :]


# speedup vs baseline: 13.7942x; 1.1667x over previous
import functools

import jax
import jax.numpy as jnp
from jax import lax
from jax.experimental import pallas as pl
from jax.experimental.pallas import tpu as pltpu

_B = 4
_N = 20000
_ROWS = 160
_LANES = 128
_PADN = _ROWS * _LANES
_MAX_DET = 100
_IOU_T = 0.2
_SCORE_T = 0.2
_NEG = float("-inf")


def _nms_body(a0, a1, a2, a3, dy, dx, dh, dw, cl,
              ox1, oy1, ox2, oy2, osc,
              x1r, y1r, x2r, y2r, arr, ssr, sor,
              *, wclip, hclip):
    a0v = a0[...]
    a1v = a1[...]
    a2v = a2[...]
    a3v = a3[...]
    ya = ((a0v + a2v) / 2.0)[None]
    xa = ((a1v + a3v) / 2.0)[None]
    ha = (a2v - a0v)[None]
    wa = (a3v - a1v)[None]
    h = jnp.exp(dh[...]) * ha
    w = jnp.exp(dw[...]) * wa
    yc = dy[...] * ha + ya
    xc = dx[...] * wa + xa
    xmin = jnp.clip(xc - w / 2.0, 0.0, wclip)
    ymin = jnp.clip(yc - h / 2.0, 0.0, hclip)
    xmax = jnp.clip(xc + w / 2.0, 0.0, wclip)
    ymax = jnp.clip(yc + h / 2.0, 0.0, hclip)
    s_orig = jax.nn.sigmoid(cl[...])
    s0 = jnp.where(s_orig > _SCORE_T, s_orig, _NEG)
    x1r[...] = xmin
    y1r[...] = ymin
    x2r[...] = xmax
    y2r[...] = ymax
    arr[...] = (xmax - xmin) * (ymax - ymin)
    ssr[...] = s0
    sor[...] = s_orig

    rowi = lax.broadcasted_iota(jnp.int32, (_ROWS, 1), 0)
    lane = lax.broadcasted_iota(jnp.int32, (1, _LANES), 1)
    big = jnp.int32(2**30)

    def body(i, _):
        hit = lane == i
        for b in range(_B):
            s = ssr[b]
            rowmax = jnp.max(s, axis=1, keepdims=True)
            m = jnp.max(rowmax)
            row = jnp.min(jnp.where(rowmax == m, rowi, big))
            srow = ssr[b, pl.ds(row, 1), :]
            lidx = jnp.min(jnp.where(srow == m, lane, big))
            lhit = lane == lidx
            bx1 = jnp.max(jnp.where(lhit, x1r[b, pl.ds(row, 1), :], _NEG))
            by1 = jnp.max(jnp.where(lhit, y1r[b, pl.ds(row, 1), :], _NEG))
            bx2 = jnp.max(jnp.where(lhit, x2r[b, pl.ds(row, 1), :], _NEG))
            by2 = jnp.max(jnp.where(lhit, y2r[b, pl.ds(row, 1), :], _NEG))
            bar = jnp.max(jnp.where(lhit, arr[b, pl.ds(row, 1), :], _NEG))
            bsc = jnp.max(jnp.where(lhit, sor[b, pl.ds(row, 1), :], _NEG))
            xx1 = jnp.maximum(bx1, x1r[b])
            yy1 = jnp.maximum(by1, y1r[b])
            xx2 = jnp.minimum(bx2, x2r[b])
            yy2 = jnp.minimum(by2, y2r[b])
            inter = jnp.maximum(xx2 - xx1, 0.0) * jnp.maximum(yy2 - yy1, 0.0)
            union = arr[b] + bar - inter
            iou = inter / jnp.maximum(union, 1e-8)
            ssr[b] = jnp.where(iou > _IOU_T, _NEG, s)
            valid = bsc > _SCORE_T
            ox1[b] = jnp.where(hit, jnp.where(valid, bx1, 0.0), ox1[b])
            oy1[b] = jnp.where(hit, jnp.where(valid, by1, 0.0), oy1[b])
            ox2[b] = jnp.where(hit, jnp.where(valid, bx2, 0.0), ox2[b])
            oy2[b] = jnp.where(hit, jnp.where(valid, by2, 0.0), oy2[b])
            osc[b] = jnp.where(hit, jnp.where(valid, bsc, 0.0), osc[b])
        return 0

    lax.fori_loop(0, _MAX_DET, body, 0)


@jax.jit
def kernel(imgs, anchors, regression, classification):
    hc = float(imgs.shape[2] - 1)
    wc = float(imgs.shape[3] - 1)
    pad = _PADN - _N
    anc = jnp.pad(anchors, ((0, pad), (0, 0)))
    reg = jnp.pad(regression, ((0, 0), (0, pad), (0, 0)))
    cls = jnp.pad(classification[..., 0], ((0, 0), (0, pad)),
                  constant_values=-1e9)
    a0, a1, a2, a3 = [anc[:, i].reshape(_ROWS, _LANES) for i in range(4)]
    dy, dx, dh, dw = [reg[..., i].reshape(_B, _ROWS, _LANES) for i in range(4)]
    cl = cls.reshape(_B, _ROWS, _LANES)

    outs = pl.pallas_call(
        functools.partial(_nms_body, wclip=wc, hclip=hc),
        out_shape=[jax.ShapeDtypeStruct((_B, 1, _LANES), jnp.float32)] * 5,
        scratch_shapes=[pltpu.VMEM((_B, _ROWS, _LANES), jnp.float32)] * 7,
    )(a0, a1, a2, a3, dy, dx, dh, dw, cl)
    ox1, oy1, ox2, oy2, osc = outs
    out = jnp.stack([ox1, oy1, ox2, oy2, osc], axis=-1)
    return out[:, 0, :_MAX_DET, :]
